# baseline (speedup 1.0000x reference)
import functools

import jax
import jax.numpy as jnp
import numpy as np
from jax import lax
from jax.experimental import pallas as pl
from jax.experimental.pallas import tpu as pltpu
from jax.experimental.pallas import tpu_sc as plsc

NC = 2
NS = 16
NW = NC * NS
LANES = 16

C = 128
W = C // 2
K = 32
NG = 8
WPG = W // NG
NQ = NW // NG
SCN = 256

_HI = np.int32(-65536)


def _sc_gather_max(x1d, idx1d, n_pad):
    npq = n_pad // NQ
    nsc = npq // SCN
    tabw = n_pad * WPG
    tabp = tabw + 16
    qw = npq * K
    mesh = plsc.VectorSubcoreMesh(core_axis_name="c", subcore_axis_name="s")

    @functools.partial(
        pl.kernel,
        out_type=jax.ShapeDtypeStruct((NG * tabw,), jnp.int32),
        mesh=mesh,
        scratch_types=[
            pltpu.VMEM((tabp,), jnp.int32),
            pltpu.VMEM((SCN * K,), jnp.int32),
            pltpu.VMEM((SCN * WPG,), jnp.int32),
        ],
    )
    def k(x_hbm, idx_hbm, agg_hbm, tab_v, idx_v, out_v):
        wid = lax.axis_index("s") * NC + lax.axis_index("c")
        g = wid >> 2
        q = wid & 3
        pltpu.sync_copy(x_hbm.at[pl.ds(g * tabp, tabp)], tab_v)
        lane = lax.iota(jnp.int32, 16)
        low8 = lane < 8

        def node_acc(iv0, iv1, base_off):
            his = [None] * 4
            los = [None] * 4
            for j in range(LANES):
                for t, iv in ((0, iv0), (1, iv1)):
                    s = iv[j]
                    d = tab_v[pl.ds(s * WPG + base_off, LANES)]
                    a = (j & 1) * 2 + t
                    if his[a] is None:
                        his[a] = d
                        los[a] = d << 16
                    else:
                        his[a] = jnp.maximum(his[a], d)
                        los[a] = jnp.maximum(los[a], d << 16)
            hi = jnp.maximum(jnp.maximum(his[0], his[1]),
                             jnp.maximum(his[2], his[3]))
            lo = jnp.maximum(jnp.maximum(los[0], los[1]),
                             jnp.maximum(los[2], los[3]))
            return (hi & _HI) | lax.shift_right_logical(lo, 16)

        @pl.loop(0, nsc)
        def _sc(s):
            pltpu.sync_copy(
                idx_hbm.at[pl.ds(q * qw + s * (SCN * K), SCN * K)], idx_v)

            @pl.loop(0, SCN // 2)
            def _pair(m):
                ia0 = idx_v[pl.ds(m * (2 * K), LANES)]
                ia1 = idx_v[pl.ds(m * (2 * K) + 16, LANES)]
                ib0 = idx_v[pl.ds(m * (2 * K) + 32, LANES)]
                ib1 = idx_v[pl.ds(m * (2 * K) + 48, LANES)]
                wa = node_acc(ia0, ia1, 8)
                wb = node_acc(ib0, ib1, 0)
                out_v[pl.ds(m * (2 * WPG), LANES)] = jnp.where(low8, wa, wb)

            pltpu.sync_copy(
                out_v.at[pl.ds(0, SCN * WPG)],
                agg_hbm.at[pl.ds(g * tabw + q * npq * WPG + s * SCN * WPG,
                                 SCN * WPG)])

    return k(x1d, idx1d)


def _tc_conv(agg, w, b2, n):
    def body(agg_ref, w_ref, b_ref, out_ref):
        prod = lax.dot_general(
            w_ref[...], agg_ref[...].astype(jnp.float32),
            (((1,), (1,)), ((), ())), preferred_element_type=jnp.float32)
        out_ref[...] = jnp.maximum(prod + b_ref[...], 0.0)

    return pl.pallas_call(
        body,
        out_shape=jax.ShapeDtypeStruct((C, n), jnp.float32),
    )(agg[:n], w, b2)


def _encode_keys(x_flat, n_pad):
    b = lax.bitcast_convert_type(x_flat.astype(jnp.bfloat16), jnp.uint16)
    neg = (b & np.uint16(0x8000)) != 0
    key = jnp.where(neg, ~b, b | np.uint16(0x8000))
    kp = (key ^ np.uint16(0x8000)).astype(jnp.uint32)
    words = (kp[:, 0::2] << 16) | kp[:, 1::2]
    words = lax.bitcast_convert_type(words, jnp.int32)
    return jnp.zeros((n_pad, W), jnp.int32).at[: x_flat.shape[0]].set(words)


def _decode_keys(agg_words, n_pad):
    u = lax.bitcast_convert_type(agg_words, jnp.uint32)
    kp = jnp.stack([u >> 16, u & np.uint32(0xFFFF)], axis=-1)
    key = kp.astype(jnp.uint16) ^ np.uint16(0x8000)
    pos = (key & np.uint16(0x8000)) != 0
    b = jnp.where(pos, key ^ np.uint16(0x8000), ~key)
    return lax.bitcast_convert_type(b, jnp.bfloat16).reshape(n_pad, C)


def kernel(x, idx, conv_w, conv_b):
    B_, C_, N_ = x.shape
    n_pad = ((N_ + NQ * SCN - 1) // (NQ * SCN)) * (NQ * SCN)
    x_flat = jnp.transpose(x, (0, 2, 1)).reshape(N_ * B_, C_)
    x_packed = _encode_keys(x_flat, n_pad)
    x1d = jnp.transpose(
        x_packed.reshape(n_pad, NG, WPG), (1, 0, 2)).reshape(NG, n_pad * WPG)
    x1d = jnp.pad(x1d, ((0, 0), (8, 8))).reshape(-1)
    idx_pad = jnp.zeros((n_pad * K,), jnp.int32).at[: idx.shape[0]].set(idx)
    agg1d = _sc_gather_max(x1d, idx_pad, n_pad)
    agg_words = jnp.transpose(
        agg1d.reshape(NG, n_pad, WPG), (1, 0, 2)).reshape(n_pad, W)
    agg = _decode_keys(agg_words, n_pad)
    out = _tc_conv(agg, conv_w, conv_b.reshape(C_, 1), N_)
    return out.reshape(B_, conv_w.shape[0], N_)


# speedup vs baseline: 1.1375x; 1.1375x over previous
import functools

import jax
import jax.numpy as jnp
import numpy as np
from jax import lax
from jax.experimental import pallas as pl
from jax.experimental.pallas import tpu as pltpu
from jax.experimental.pallas import tpu_sc as plsc

NC = 2
NS = 16
NW = NC * NS
LANES = 16

C = 128
W = C // 2
K = 32
NG = 8
WPG = W // NG
NQ = NW // NG
SCN = 256

_HI = np.int32(-65536)


def _sc_gather_max(x1d, idx1d, n_pad):
    npq = n_pad // NQ
    nsc = npq // SCN
    tabw = n_pad * WPG
    tabp = tabw + 16
    qw = npq * K
    mesh = plsc.VectorSubcoreMesh(core_axis_name="c", subcore_axis_name="s")

    @functools.partial(
        pl.kernel,
        out_type=jax.ShapeDtypeStruct((NG * tabw,), jnp.int32),
        mesh=mesh,
        scratch_types=[
            pltpu.VMEM((tabp,), jnp.int32),
            pltpu.VMEM((SCN * K,), jnp.int32),
            pltpu.VMEM((SCN * WPG,), jnp.int32),
        ],
    )
    def k(x_hbm, idx_hbm, agg_hbm, tab_v, idx_v, out_v):
        wid = lax.axis_index("s") * NC + lax.axis_index("c")
        g = wid >> 2
        q = wid & 3
        pltpu.sync_copy(x_hbm.at[pl.ds(g * tabp, tabp)], tab_v)
        lane = lax.iota(jnp.int32, 16)
        low8 = lane < 8

        def node_acc(iv0, iv1, base_off):
            his = [None] * 4
            los = [None] * 4
            for j in range(LANES):
                for t, iv in ((0, iv0), (1, iv1)):
                    s = iv[j]
                    d = tab_v[pl.ds(s * WPG + base_off, LANES)]
                    a = (j & 1) * 2 + t
                    if his[a] is None:
                        his[a] = d
                        los[a] = d << 16
                    else:
                        his[a] = jnp.maximum(his[a], d)
                        los[a] = jnp.maximum(los[a], d << 16)
            hi = jnp.maximum(jnp.maximum(his[0], his[1]),
                             jnp.maximum(his[2], his[3]))
            lo = jnp.maximum(jnp.maximum(los[0], los[1]),
                             jnp.maximum(los[2], los[3]))
            return (hi & _HI) | lax.shift_right_logical(lo, 16)

        @pl.loop(0, nsc)
        def _sc(s):
            pltpu.sync_copy(
                idx_hbm.at[pl.ds(q * qw + s * (SCN * K), SCN * K)], idx_v)

            @pl.loop(0, SCN // 2)
            def _pair(m):
                ia0 = idx_v[pl.ds(m * (2 * K), LANES)]
                ia1 = idx_v[pl.ds(m * (2 * K) + 16, LANES)]
                ib0 = idx_v[pl.ds(m * (2 * K) + 32, LANES)]
                ib1 = idx_v[pl.ds(m * (2 * K) + 48, LANES)]
                wa = node_acc(ia0, ia1, 8)
                wb = node_acc(ib0, ib1, 0)
                out_v[pl.ds(m * (2 * WPG), LANES)] = jnp.where(low8, wa, wb)

            pltpu.sync_copy(
                out_v.at[pl.ds(0, SCN * WPG)],
                agg_hbm.at[pl.ds(g * tabw + q * npq * WPG + s * SCN * WPG,
                                 SCN * WPG)])

    return k(x1d, idx1d)


def _tc_conv(agg, w, b2, n):
    def body(agg_ref, w_ref, b_ref, out_ref):
        prod = lax.dot_general(
            w_ref[...], agg_ref[...].astype(jnp.float32),
            (((1,), (1,)), ((), ())), preferred_element_type=jnp.float32)
        out_ref[...] = jnp.maximum(prod + b_ref[...], 0.0)

    return pl.pallas_call(
        body,
        out_shape=jax.ShapeDtypeStruct((C, n), jnp.float32),
    )(agg[:n], w, b2)


def _key_xform(words_i32):
    u = lax.bitcast_convert_type(words_i32, jnp.uint32)
    m = ((u & np.uint32(0x80008000)) >> 15) * np.uint32(0x7FFF)
    return lax.bitcast_convert_type(u ^ m, jnp.int32)


def _encode_keys(x_flat, n_pad):
    words = lax.bitcast_convert_type(
        x_flat.astype(jnp.bfloat16).reshape(x_flat.shape[0], W, 2), jnp.int32)
    enc = _key_xform(words)
    return jnp.zeros((n_pad, W), jnp.int32).at[: x_flat.shape[0]].set(enc)


def _decode_keys(agg_words, n_pad):
    words = _key_xform(agg_words)
    return lax.bitcast_convert_type(
        words, jnp.bfloat16).reshape(n_pad, C)


def kernel(x, idx, conv_w, conv_b):
    B_, C_, N_ = x.shape
    n_pad = ((N_ + NQ * SCN - 1) // (NQ * SCN)) * (NQ * SCN)
    x_flat = jnp.transpose(x, (0, 2, 1)).reshape(N_ * B_, C_)
    x_packed = _encode_keys(x_flat, n_pad)
    x1d = jnp.transpose(
        x_packed.reshape(n_pad, NG, WPG), (1, 0, 2)).reshape(NG, n_pad * WPG)
    x1d = jnp.pad(x1d, ((0, 0), (8, 8))).reshape(-1)
    idx_pad = jnp.zeros((n_pad * K,), jnp.int32).at[: idx.shape[0]].set(idx)
    agg1d = _sc_gather_max(x1d, idx_pad, n_pad)
    agg_words = jnp.transpose(
        agg1d.reshape(NG, n_pad, WPG), (1, 0, 2)).reshape(n_pad, W)
    agg = _decode_keys(agg_words, n_pad)
    out = _tc_conv(agg, conv_w, conv_b.reshape(C_, 1), N_)
    return out.reshape(B_, conv_w.shape[0], N_)
